# baseline (speedup 1.0000x reference)
import functools

import jax
import jax.numpy as jnp
from jax import lax
from jax.experimental import pallas as pl
from jax.experimental.pallas import tpu as pltpu
from jax.experimental.pallas import tpu_sc as plsc

B = 2
N = 10000
E = 160000
C = 128
O = 128
K = 8

NS = 16
CW = 96
SUB = CW // 16
EPB = E + N
CH = 112
EPAD = NS * CH * CW
RP = 640

_BLK = 2000
_GRID = (B * N) // _BLK


def _prep_body(x_ref, w_ref, ws_ref, wd_ref, h_ref, as_ref, ad_ref, gm_ref):
    i = pl.program_id(0)
    h = jnp.dot(x_ref[...], w_ref[...], preferred_element_type=jnp.float32)
    h_ref[...] = h
    a_s = jnp.sum(h * ws_ref[...], axis=1, keepdims=True)
    a_d = jnp.sum(h * wd_ref[...], axis=1, keepdims=True)
    as_ref[...] = a_s
    ad_ref[...] = a_d
    bm = jnp.max(a_s).reshape(1, 1)

    @pl.when(i == 0)
    def _():
        gm_ref[...] = bm

    @pl.when(i > 0)
    def _():
        gm_ref[...] = jnp.maximum(gm_ref[...], bm)


def _prep(x_flat, W, ws, wd):
    return pl.pallas_call(
        _prep_body,
        grid=(_GRID,),
        in_specs=[
            pl.BlockSpec((_BLK, C), lambda i: (i, 0)),
            pl.BlockSpec((C, O), lambda i: (0, 0)),
            pl.BlockSpec((1, O), lambda i: (0, 0)),
            pl.BlockSpec((1, O), lambda i: (0, 0)),
        ],
        out_specs=[
            pl.BlockSpec((_BLK, O), lambda i: (i, 0)),
            pl.BlockSpec((_BLK, 1), lambda i: (i, 0)),
            pl.BlockSpec((_BLK, 1), lambda i: (i, 0)),
            pl.BlockSpec((1, 1), lambda i: (0, 0)),
        ],
        out_shape=[
            jax.ShapeDtypeStruct((B * N, O), jnp.float32),
            jax.ShapeDtypeStruct((B * N, 1), jnp.float32),
            jax.ShapeDtypeStruct((B * N, 1), jnp.float32),
            jax.ShapeDtypeStruct((1, 1), jnp.float32),
        ],
    )(x_flat, W, ws, wd)


def _gat_sc_body(h_hbm, asrc_hbm, adst_hbm, gmax_hbm, srcg_hbm, dstl_hbm,
                 zeros_hbm, zerod_hbm, acc_hbm, den_hbm,
                 acc_s, den_s, asrc_t, adst_t,
                 stA, dtA, dscA, pbA, hrA,
                 stB, dtB, dscB, pbB, hrB, g_t,
                 semGA, semGB, semIA, semIB, semSA, semSB):
    c = lax.axis_index("c")
    s = lax.axis_index("s")
    pltpu.sync_copy(asrc_hbm.at[c], asrc_t)
    pltpu.sync_copy(adst_hbm.at[c], adst_t)
    pltpu.sync_copy(gmax_hbm, g_t)
    rb = pl.multiple_of(jnp.minimum(s * RP, N - RP), 8)
    pltpu.sync_copy(zeros_hbm.at[pl.ds(rb, RP)], acc_s.at[pl.ds(rb, RP)])
    pltpu.sync_copy(zerod_hbm.at[pl.ds(rb, RP)], den_s.at[pl.ds(rb, RP)])
    plsc.subcore_barrier()

    g = g_t[...]
    lane = lax.broadcasted_iota(jnp.int32, (16,), 0)
    coff = c * N

    pltpu.sync_copy(srcg_hbm.at[c, s, 0], stA)
    pltpu.sync_copy(dstl_hbm.at[s, 0], dtA)
    pltpu.sync_copy(srcg_hbm.at[c, s, 1], stB)
    pltpu.sync_copy(dstl_hbm.at[s, 1], dtB)
    pltpu.async_copy(h_hbm.at[stA], hrA, semGA)

    def half(j, i, first,
             st, dt, dsc, pb, hr, semG, semI, semS,
             stn, dtn, dscn, pbn, hrn, semGn, semIn, semSn):
        base = (s * CH + j) * CW
        for k in range(SUB):
            srcv = st[pl.ds(k * 16, 16)] - coff
            dstv = dt[pl.ds(k * 16, 16)]
            a_s = plsc.load_gather(asrc_t, [srcv])
            a_d = plsc.load_gather(adst_t, [dstv])
            t16 = a_s + a_d
            e16 = jnp.maximum(t16, 0.2 * t16)
            mg = g + a_d
            m16 = jnp.maximum(mg, 0.2 * mg)
            gi = base + k * 16 + lane
            pb[pl.ds(k * 16, 16)] = jnp.where(gi < EPB, jnp.exp(e16 - m16), 0.0)
            dsc[pl.ds(k * 16, 16)] = dstv
        pltpu.make_async_copy(h_hbm.at[st], hr, semG).wait()
        jn = jnp.minimum(j + 2, CH - 1)
        pltpu.async_copy(srcg_hbm.at[c, s, jn], st, semI)
        pltpu.async_copy(dstl_hbm.at[s, jn], dt, semI)

        def issue_next():
            pltpu.make_async_copy(hrn, acc_s.at[dscn], semSn).wait()
            pltpu.make_async_copy(pbn, den_s.at[dscn], semSn).wait()
            pltpu.make_async_copy(srcg_hbm.at[c, s, 0], stn, semIn).wait()
            pltpu.make_async_copy(dstl_hbm.at[s, 0], dtn, semIn).wait()
            pltpu.async_copy(h_hbm.at[stn], hrn, semGn)

        if first:
            @pl.when(i > 0)
            def _():
                issue_next()

            @pl.when(i == 0)
            def _():
                pltpu.async_copy(h_hbm.at[stn], hrn, semGn)
        else:
            @pl.when(j + 1 < CH)
            def _():
                issue_next()

        @plsc.parallel_loop(0, CW, unroll=4)
        def _(r):
            pbv = plsc.load_gather(pb, [jnp.full((16,), r, jnp.int32)])
            for k in range(8):
                hr[r, pl.ds(k * 16, 16)] = hr[r, pl.ds(k * 16, 16)] * pbv

        pltpu.async_copy(hr, acc_s.at[dsc], semS, add=True)
        pltpu.async_copy(pb, den_s.at[dsc], semS, add=True)

    def body(i, carry):
        half(2 * i, i, True,
             stA, dtA, dscA, pbA, hrA, semGA, semIA, semSA,
             stB, dtB, dscB, pbB, hrB, semGB, semIB, semSB)
        half(2 * i + 1, i, False,
             stB, dtB, dscB, pbB, hrB, semGB, semIB, semSB,
             stA, dtA, dscA, pbA, hrA, semGA, semIA, semSA)
        return carry

    lax.fori_loop(0, CH // 2, body, 0)
    pltpu.make_async_copy(hrA, acc_s.at[dscA], semSA).wait()
    pltpu.make_async_copy(pbA, den_s.at[dscA], semSA).wait()
    pltpu.make_async_copy(hrB, acc_s.at[dscB], semSB).wait()
    pltpu.make_async_copy(pbB, den_s.at[dscB], semSB).wait()
    pltpu.make_async_copy(srcg_hbm.at[c, s, 0], stA, semIA).wait()
    pltpu.make_async_copy(dstl_hbm.at[s, 0], dtA, semIA).wait()
    pltpu.make_async_copy(srcg_hbm.at[c, s, 0], stB, semIB).wait()
    pltpu.make_async_copy(dstl_hbm.at[s, 0], dtB, semIB).wait()
    plsc.subcore_barrier()
    pltpu.sync_copy(acc_s.at[pl.ds(rb, RP)], acc_hbm.at[c, pl.ds(rb, RP)])
    pltpu.sync_copy(den_s.at[pl.ds(rb, RP)], den_hbm.at[c, pl.ds(rb, RP)])


def _gat_sc(h, asrc2, adst2, gmax16, srcg, dstl, zeros, zerod):
    mesh = plsc.VectorSubcoreMesh(core_axis_name="c", subcore_axis_name="s")
    idx = lambda: pltpu.VMEM((CW,), jnp.int32)
    fn = functools.partial(
        pl.kernel,
        mesh=mesh,
        out_type=(jax.ShapeDtypeStruct((B, N, O), jnp.float32),
                  jax.ShapeDtypeStruct((B, N), jnp.float32)),
        scratch_types=[
            pltpu.VMEM_SHARED((N, O), jnp.float32),
            pltpu.VMEM_SHARED((N,), jnp.float32),
            pltpu.VMEM((N,), jnp.float32),
            pltpu.VMEM((N,), jnp.float32),
            idx(), idx(), idx(), pltpu.VMEM((CW,), jnp.float32),
            pltpu.VMEM((CW, O), jnp.float32),
            idx(), idx(), idx(), pltpu.VMEM((CW,), jnp.float32),
            pltpu.VMEM((CW, O), jnp.float32),
            pltpu.VMEM((16,), jnp.float32),
            pltpu.SemaphoreType.DMA, pltpu.SemaphoreType.DMA,
            pltpu.SemaphoreType.DMA, pltpu.SemaphoreType.DMA,
            pltpu.SemaphoreType.DMA, pltpu.SemaphoreType.DMA,
        ],
        compiler_params=pltpu.CompilerParams(needs_layout_passes=False,
                                             use_tc_tiling_on_sc=False),
    )(_gat_sc_body)
    return fn(h, asrc2, adst2, gmax16, srcg, dstl, zeros, zerod)


def _dec_body(acc_ref, den_ref, tgt_ref, bias_ref, wdd_ref, bd_ref,
              val_ref, idx_ref, nll_ref, cnt_ref):
    i = pl.program_id(0)
    a = acc_ref[...]
    den = den_ref[...]
    out = a / (den + 1e-16) + bias_ref[...]
    logits = jnp.dot(out.astype(jnp.bfloat16), wdd_ref[...].astype(jnp.bfloat16),
                     preferred_element_type=jnp.float32)
    logits = logits + bd_ref[...]
    mx = jnp.max(logits, axis=1, keepdims=True)
    val_ref[...] = mx
    iota = lax.broadcasted_iota(jnp.int32, logits.shape, 1)
    idx_ref[...] = jnp.min(jnp.where(logits == mx, iota, K), axis=1,
                           keepdims=True)
    logz = mx[:, 0] + jnp.log(jnp.sum(jnp.exp(logits - mx), axis=1))
    t = tgt_ref[...][:, 0]
    valid = t >= 0
    safe = jnp.where(valid, t, 0)
    picked = jnp.sum(jnp.where(iota == safe[:, None], logits, 0.0), axis=1)
    nll = jnp.sum(jnp.where(valid, logz - picked, 0.0)).reshape(1, 1)
    vc = jnp.sum(valid.astype(jnp.float32)).reshape(1, 1)

    @pl.when(i == 0)
    def _():
        nll_ref[...] = nll
        cnt_ref[...] = vc

    @pl.when(i > 0)
    def _():
        nll_ref[...] = nll_ref[...] + nll
        cnt_ref[...] = cnt_ref[...] + vc


def _decode(acc_flat, den_flat, tgt, bias, Wd, bd):
    return pl.pallas_call(
        _dec_body,
        grid=(_GRID,),
        in_specs=[
            pl.BlockSpec((_BLK, O), lambda i: (i, 0)),
            pl.BlockSpec((_BLK, 1), lambda i: (i, 0)),
            pl.BlockSpec((_BLK, 1), lambda i: (i, 0)),
            pl.BlockSpec((1, O), lambda i: (0, 0)),
            pl.BlockSpec((O, K), lambda i: (0, 0)),
            pl.BlockSpec((1, K), lambda i: (0, 0)),
        ],
        out_specs=[
            pl.BlockSpec((_BLK, 1), lambda i: (i, 0)),
            pl.BlockSpec((_BLK, 1), lambda i: (i, 0)),
            pl.BlockSpec((1, 1), lambda i: (0, 0)),
            pl.BlockSpec((1, 1), lambda i: (0, 0)),
        ],
        out_shape=[
            jax.ShapeDtypeStruct((B * N, 1), jnp.float32),
            jax.ShapeDtypeStruct((B * N, 1), jnp.int32),
            jax.ShapeDtypeStruct((1, 1), jnp.float32),
            jax.ShapeDtypeStruct((1, 1), jnp.float32),
        ],
    )(acc_flat, den_flat, tgt, bias, Wd, bd)


def kernel(x, adjacency, targets, W, att_src, att_dst, bias, Wd, bd):
    x_flat = x.reshape(B * N, C)
    ws = att_src.reshape(1, O)
    wd = att_dst.reshape(1, O)
    h, asrc, adst, gmax = _prep(x_flat, W, ws, wd)

    sl = jnp.arange(N, dtype=jnp.int32)
    src = jnp.concatenate([adjacency[0].astype(jnp.int32), sl])
    dst = jnp.concatenate([adjacency[1].astype(jnp.int32), sl])
    src = jnp.pad(src, (0, EPAD - EPB))
    dst = jnp.pad(dst, (0, EPAD - EPB))
    srcg = jnp.stack([src, src + N]).reshape(B, NS, CH, CW)
    dstl = dst.reshape(NS, CH, CW)

    asrc2 = asrc.reshape(B, N)
    adst2 = adst.reshape(B, N)
    gmax16 = jnp.broadcast_to(gmax.reshape(1), (16,))
    zeros = jnp.zeros((N, O), jnp.float32)
    zerod = jnp.zeros((N,), jnp.float32)

    acc, den = _gat_sc(h, asrc2, adst2, gmax16, srcg, dstl, zeros, zerod)

    vals, idxs, nll, cnt = _decode(acc.reshape(B * N, O),
                                   den.reshape(B * N, 1),
                                   targets.reshape(B * N, 1).astype(jnp.int32),
                                   bias.reshape(1, O), Wd, bd.reshape(1, K))
    loss = (nll[0, 0] / jnp.maximum(cnt[0, 0], 1.0)) / B
    return (vals.reshape(B, N, 1), idxs.reshape(B, N, 1), loss)


# speedup vs baseline: 1.0221x; 1.0221x over previous
import functools

import jax
import jax.numpy as jnp
from jax import lax
from jax.experimental import pallas as pl
from jax.experimental.pallas import tpu as pltpu
from jax.experimental.pallas import tpu_sc as plsc

B = 2
N = 10000
E = 160000
C = 128
O = 128
K = 8

NS = 16
CW = 96
SUB = CW // 16
EPB = E + N
CH = 112
EPAD = NS * CH * CW
RP = 640

_BLK = 5000
_GRID = (B * N) // _BLK


def _prep_body(x_ref, w_ref, ws_ref, wd_ref, h_ref, as_ref, ad_ref, gm_ref):
    i = pl.program_id(0)
    h = jnp.dot(x_ref[...], w_ref[...], preferred_element_type=jnp.float32)
    h_ref[...] = h
    a_s = jnp.sum(h * ws_ref[...], axis=1, keepdims=True)
    a_d = jnp.sum(h * wd_ref[...], axis=1, keepdims=True)
    as_ref[...] = a_s
    ad_ref[...] = a_d
    bm = jnp.max(a_s).reshape(1, 1)

    @pl.when(i == 0)
    def _():
        gm_ref[...] = bm

    @pl.when(i > 0)
    def _():
        gm_ref[...] = jnp.maximum(gm_ref[...], bm)


def _prep(x_flat, W, ws, wd):
    return pl.pallas_call(
        _prep_body,
        grid=(_GRID,),
        in_specs=[
            pl.BlockSpec((_BLK, C), lambda i: (i, 0)),
            pl.BlockSpec((C, O), lambda i: (0, 0)),
            pl.BlockSpec((1, O), lambda i: (0, 0)),
            pl.BlockSpec((1, O), lambda i: (0, 0)),
        ],
        out_specs=[
            pl.BlockSpec((_BLK, O), lambda i: (i, 0)),
            pl.BlockSpec((_BLK, 1), lambda i: (i, 0)),
            pl.BlockSpec((_BLK, 1), lambda i: (i, 0)),
            pl.BlockSpec((1, 1), lambda i: (0, 0)),
        ],
        out_shape=[
            jax.ShapeDtypeStruct((B * N, O), jnp.float32),
            jax.ShapeDtypeStruct((B * N, 1), jnp.float32),
            jax.ShapeDtypeStruct((B * N, 1), jnp.float32),
            jax.ShapeDtypeStruct((1, 1), jnp.float32),
        ],
    )(x_flat, W, ws, wd)


def _gat_sc_body(h_hbm, asrc_hbm, adst_hbm, gmax_hbm, srcg_hbm, dstl_hbm,
                 acc_hbm, den_hbm,
                 acc_s, den_s, asrc_t, adst_t,
                 stA, dtA, dscA, pbA, hrA,
                 stB, dtB, dscB, pbB, hrB, g_t,
                 semGA, semGB, semIA, semIB, semSA, semSB):
    c = lax.axis_index("c")
    s = lax.axis_index("s")
    pltpu.sync_copy(asrc_hbm.at[c], asrc_t)
    pltpu.sync_copy(adst_hbm.at[c], adst_t)
    pltpu.sync_copy(gmax_hbm, g_t)
    rb = pl.multiple_of(jnp.minimum(s * RP, N - RP), 8)

    zv = jnp.zeros((16,), jnp.float32)

    @plsc.parallel_loop(0, CW, unroll=4)
    def _(r):
        for k in range(8):
            hrA[r, pl.ds(k * 16, 16)] = zv

    for k in range(SUB):
        pbA[pl.ds(k * 16, 16)] = zv
    for q in range(RP // CW):
        pltpu.sync_copy(hrA, acc_s.at[pl.ds(rb + q * CW, CW)])
        pltpu.sync_copy(pbA, den_s.at[pl.ds(rb + q * CW, CW)])
    _TAIL = RP - (RP // CW) * CW
    pltpu.sync_copy(hrA.at[pl.ds(0, _TAIL)],
                    acc_s.at[pl.ds(rb + RP - _TAIL, _TAIL)])
    pltpu.sync_copy(pbA.at[pl.ds(0, _TAIL)],
                    den_s.at[pl.ds(rb + RP - _TAIL, _TAIL)])
    plsc.subcore_barrier()

    g = g_t[...]
    lane = lax.broadcasted_iota(jnp.int32, (16,), 0)
    coff = c * N

    pltpu.sync_copy(srcg_hbm.at[c, s, 0], stA)
    pltpu.sync_copy(dstl_hbm.at[s, 0], dtA)
    pltpu.sync_copy(srcg_hbm.at[c, s, 1], stB)
    pltpu.sync_copy(dstl_hbm.at[s, 1], dtB)
    pltpu.async_copy(h_hbm.at[stA], hrA, semGA)

    def half(j, i, first,
             st, dt, dsc, pb, hr, semG, semI, semS,
             stn, dtn, dscn, pbn, hrn, semGn, semIn, semSn):
        base = (s * CH + j) * CW
        for k in range(SUB):
            srcv = st[pl.ds(k * 16, 16)] - coff
            dstv = dt[pl.ds(k * 16, 16)]
            a_s = plsc.load_gather(asrc_t, [srcv])
            a_d = plsc.load_gather(adst_t, [dstv])
            t16 = a_s + a_d
            e16 = jnp.maximum(t16, 0.2 * t16)
            mg = g + a_d
            m16 = jnp.maximum(mg, 0.2 * mg)
            gi = base + k * 16 + lane
            pb[pl.ds(k * 16, 16)] = jnp.where(gi < EPB, jnp.exp(e16 - m16), 0.0)
            dsc[pl.ds(k * 16, 16)] = dstv
        pltpu.make_async_copy(h_hbm.at[st], hr, semG).wait()
        jn = jnp.minimum(j + 2, CH - 1)
        pltpu.async_copy(srcg_hbm.at[c, s, jn], st, semI)
        pltpu.async_copy(dstl_hbm.at[s, jn], dt, semI)

        def issue_next():
            pltpu.make_async_copy(hrn, acc_s.at[dscn], semSn).wait()
            pltpu.make_async_copy(pbn, den_s.at[dscn], semSn).wait()
            pltpu.make_async_copy(srcg_hbm.at[c, s, 0], stn, semIn).wait()
            pltpu.make_async_copy(dstl_hbm.at[s, 0], dtn, semIn).wait()
            pltpu.async_copy(h_hbm.at[stn], hrn, semGn)

        if first:
            @pl.when(i > 0)
            def _():
                issue_next()

            @pl.when(i == 0)
            def _():
                pltpu.async_copy(h_hbm.at[stn], hrn, semGn)
        else:
            @pl.when(j + 1 < CH)
            def _():
                issue_next()

        @plsc.parallel_loop(0, CW, unroll=4)
        def _(r):
            pbv = plsc.load_gather(pb, [jnp.full((16,), r, jnp.int32)])
            for k in range(8):
                hr[r, pl.ds(k * 16, 16)] = hr[r, pl.ds(k * 16, 16)] * pbv

        pltpu.async_copy(hr, acc_s.at[dsc], semS, add=True)
        pltpu.async_copy(pb, den_s.at[dsc], semS, add=True)

    def body(i, carry):
        half(2 * i, i, True,
             stA, dtA, dscA, pbA, hrA, semGA, semIA, semSA,
             stB, dtB, dscB, pbB, hrB, semGB, semIB, semSB)
        half(2 * i + 1, i, False,
             stB, dtB, dscB, pbB, hrB, semGB, semIB, semSB,
             stA, dtA, dscA, pbA, hrA, semGA, semIA, semSA)
        return carry

    lax.fori_loop(0, CH // 2, body, 0)
    pltpu.make_async_copy(hrA, acc_s.at[dscA], semSA).wait()
    pltpu.make_async_copy(pbA, den_s.at[dscA], semSA).wait()
    pltpu.make_async_copy(hrB, acc_s.at[dscB], semSB).wait()
    pltpu.make_async_copy(pbB, den_s.at[dscB], semSB).wait()
    pltpu.make_async_copy(srcg_hbm.at[c, s, 0], stA, semIA).wait()
    pltpu.make_async_copy(dstl_hbm.at[s, 0], dtA, semIA).wait()
    pltpu.make_async_copy(srcg_hbm.at[c, s, 0], stB, semIB).wait()
    pltpu.make_async_copy(dstl_hbm.at[s, 0], dtB, semIB).wait()
    plsc.subcore_barrier()
    pltpu.sync_copy(acc_s.at[pl.ds(rb, RP)], acc_hbm.at[c, pl.ds(rb, RP)])
    pltpu.sync_copy(den_s.at[pl.ds(rb, RP)], den_hbm.at[c, pl.ds(rb, RP)])


def _gat_sc(h, asrc2, adst2, gmax16, srcg, dstl):
    mesh = plsc.VectorSubcoreMesh(core_axis_name="c", subcore_axis_name="s")
    idx = lambda: pltpu.VMEM((CW,), jnp.int32)
    fn = functools.partial(
        pl.kernel,
        mesh=mesh,
        out_type=(jax.ShapeDtypeStruct((B, N, O), jnp.float32),
                  jax.ShapeDtypeStruct((B, N), jnp.float32)),
        scratch_types=[
            pltpu.VMEM_SHARED((N, O), jnp.float32),
            pltpu.VMEM_SHARED((N,), jnp.float32),
            pltpu.VMEM((N,), jnp.float32),
            pltpu.VMEM((N,), jnp.float32),
            idx(), idx(), idx(), pltpu.VMEM((CW,), jnp.float32),
            pltpu.VMEM((CW, O), jnp.float32),
            idx(), idx(), idx(), pltpu.VMEM((CW,), jnp.float32),
            pltpu.VMEM((CW, O), jnp.float32),
            pltpu.VMEM((16,), jnp.float32),
            pltpu.SemaphoreType.DMA, pltpu.SemaphoreType.DMA,
            pltpu.SemaphoreType.DMA, pltpu.SemaphoreType.DMA,
            pltpu.SemaphoreType.DMA, pltpu.SemaphoreType.DMA,
        ],
        compiler_params=pltpu.CompilerParams(needs_layout_passes=False,
                                             use_tc_tiling_on_sc=False),
    )(_gat_sc_body)
    return fn(h, asrc2, adst2, gmax16, srcg, dstl)


def _dec_body(acc_ref, den_ref, tgt_ref, bias_ref, wdd_ref, bd_ref,
              val_ref, idx_ref, nll_ref, cnt_ref):
    i = pl.program_id(0)
    a = acc_ref[...]
    den = den_ref[...]
    out = a / (den + 1e-16) + bias_ref[...]
    logits = jnp.dot(out.astype(jnp.bfloat16), wdd_ref[...].astype(jnp.bfloat16),
                     preferred_element_type=jnp.float32)
    logits = logits + bd_ref[...]
    mx = jnp.max(logits, axis=1, keepdims=True)
    val_ref[...] = mx
    iota = lax.broadcasted_iota(jnp.int32, logits.shape, 1)
    idx_ref[...] = jnp.min(jnp.where(logits == mx, iota, K), axis=1,
                           keepdims=True)
    logz = mx[:, 0] + jnp.log(jnp.sum(jnp.exp(logits - mx), axis=1))
    t = tgt_ref[...][:, 0]
    valid = t >= 0
    safe = jnp.where(valid, t, 0)
    picked = jnp.sum(jnp.where(iota == safe[:, None], logits, 0.0), axis=1)
    nll = jnp.sum(jnp.where(valid, logz - picked, 0.0)).reshape(1, 1)
    vc = jnp.sum(valid.astype(jnp.float32)).reshape(1, 1)

    @pl.when(i == 0)
    def _():
        nll_ref[...] = nll
        cnt_ref[...] = vc

    @pl.when(i > 0)
    def _():
        nll_ref[...] = nll_ref[...] + nll
        cnt_ref[...] = cnt_ref[...] + vc


def _decode(acc_flat, den_flat, tgt, bias, Wd, bd):
    return pl.pallas_call(
        _dec_body,
        grid=(_GRID,),
        in_specs=[
            pl.BlockSpec((_BLK, O), lambda i: (i, 0)),
            pl.BlockSpec((_BLK, 1), lambda i: (i, 0)),
            pl.BlockSpec((_BLK, 1), lambda i: (i, 0)),
            pl.BlockSpec((1, O), lambda i: (0, 0)),
            pl.BlockSpec((O, K), lambda i: (0, 0)),
            pl.BlockSpec((1, K), lambda i: (0, 0)),
        ],
        out_specs=[
            pl.BlockSpec((_BLK, 1), lambda i: (i, 0)),
            pl.BlockSpec((_BLK, 1), lambda i: (i, 0)),
            pl.BlockSpec((1, 1), lambda i: (0, 0)),
            pl.BlockSpec((1, 1), lambda i: (0, 0)),
        ],
        out_shape=[
            jax.ShapeDtypeStruct((B * N, 1), jnp.float32),
            jax.ShapeDtypeStruct((B * N, 1), jnp.int32),
            jax.ShapeDtypeStruct((1, 1), jnp.float32),
            jax.ShapeDtypeStruct((1, 1), jnp.float32),
        ],
    )(acc_flat, den_flat, tgt, bias, Wd, bd)


def kernel(x, adjacency, targets, W, att_src, att_dst, bias, Wd, bd):
    x_flat = x.reshape(B * N, C)
    ws = att_src.reshape(1, O)
    wd = att_dst.reshape(1, O)
    h, asrc, adst, gmax = _prep(x_flat, W, ws, wd)

    sl = jnp.arange(N, dtype=jnp.int32)
    src = jnp.concatenate([adjacency[0].astype(jnp.int32), sl])
    dst = jnp.concatenate([adjacency[1].astype(jnp.int32), sl])
    src = jnp.pad(src, (0, EPAD - EPB))
    dst = jnp.pad(dst, (0, EPAD - EPB))
    srcg = jnp.stack([src, src + N]).reshape(B, NS, CH, CW)
    dstl = dst.reshape(NS, CH, CW)

    asrc2 = asrc.reshape(B, N)
    adst2 = adst.reshape(B, N)
    gmax16 = jnp.broadcast_to(gmax.reshape(1), (16,))

    acc, den = _gat_sc(h, asrc2, adst2, gmax16, srcg, dstl)

    vals, idxs, nll, cnt = _decode(acc.reshape(B * N, O),
                                   den.reshape(B * N, 1),
                                   targets.reshape(B * N, 1).astype(jnp.int32),
                                   bias.reshape(1, O), Wd, bd.reshape(1, K))
    loss = (nll[0, 0] / jnp.maximum(cnt[0, 0], 1.0)) / B
    return (vals.reshape(B, N, 1), idxs.reshape(B, N, 1), loss)
